# baseline (speedup 1.0000x reference)
import functools

import jax
import jax.numpy as jnp
from jax import lax
from jax.experimental import pallas as pl
from jax.experimental.pallas import tpu as pltpu
from jax.experimental.pallas import tpu_sc as plsc

_VP = 1024
_KP = 64
_DUMMY = 1016
_NC = 2
_NS = 16
_NW = _NC * _NS
_CH = 16


def _hist_sc_kernel(rec_hbm, c_hbm, rec_v, counts_v):
    wid = lax.axis_index("s") * _NC + lax.axis_index("c")
    n_chunks = rec_hbm.shape[0]
    per_w = n_chunks // _NW
    row_off = lax.broadcasted_iota(jnp.int32, (16,), 0) * _VP
    zeros16 = jnp.zeros((16,), jnp.float32)
    ones16 = jnp.ones((16,), jnp.float32)

    for ch in range(per_w):
        chunk = wid * per_w + ch
        pltpu.sync_copy(rec_hbm.at[chunk], rec_v)

        def zero_body(i, _):
            counts_v[pl.ds(i * 16, 16)] = zeros16
            return 0
        lax.fori_loop(0, _CH * _VP // 16, zero_body, 0, unroll=8)

        for j in range(_KP):
            ids = rec_v[pl.ds(j * 16, 16)]
            plsc.addupdate_scatter(counts_v, [ids + row_off], ones16)

        pltpu.sync_copy(counts_v, c_hbm.at[chunk])


def _qf_tc_kernel(c_ref, d_ref, out_ref):
    c = c_ref[...]
    t = jnp.dot(c, d_ref[...], preferred_element_type=jnp.float32)
    out_ref[...] = jnp.sum(t * c, axis=1)


@jax.jit
def kernel(user_sequence, recommendations, distance_matrix):
    del user_sequence
    b, k = recommendations.shape
    v = distance_matrix.shape[0]
    d_pad = jnp.zeros((_VP, _VP), jnp.float32).at[:v, :v].set(distance_matrix)

    rec = recommendations.astype(jnp.int32)
    rec_pad = jnp.full((b, _KP), _DUMMY, jnp.int32).at[:, :k].set(rec)
    rec_t = rec_pad.reshape(b // _CH, _CH, _KP).transpose(0, 2, 1)
    rec_t = rec_t.reshape(b // _CH, _KP * _CH)

    mesh = plsc.VectorSubcoreMesh(core_axis_name="c", subcore_axis_name="s")
    counts = pl.kernel(
        _hist_sc_kernel,
        out_type=jax.ShapeDtypeStruct((b // _CH, _CH * _VP), jnp.float32),
        mesh=mesh,
        compiler_params=pltpu.CompilerParams(needs_layout_passes=False),
        scratch_types=[
            pltpu.VMEM((_KP * _CH,), jnp.int32),
            pltpu.VMEM((_CH * _VP,), jnp.float32),
        ],
    )(rec_t)
    counts = counts.reshape(b, _VP)

    bblk = 256
    distance_sum = pl.pallas_call(
        _qf_tc_kernel,
        grid=(b // bblk,),
        in_specs=[
            pl.BlockSpec((bblk, _VP), lambda i: (i, 0)),
            pl.BlockSpec((_VP, _VP), lambda i: (0, 0)),
        ],
        out_specs=pl.BlockSpec((bblk,), lambda i: (i,)),
        out_shape=jax.ShapeDtypeStruct((b,), jnp.float32),
    )(counts, d_pad)
    return distance_sum / (k * (k - 1))


# speedup vs baseline: 1.4029x; 1.4029x over previous
import functools

import jax
import jax.numpy as jnp
from jax import lax
from jax.experimental import pallas as pl
from jax.experimental.pallas import tpu as pltpu
from jax.experimental.pallas import tpu_sc as plsc

_VP = 1024
_KP = 64
_DUMMY = 1016
_NC = 2
_NS = 16
_NW = _NC * _NS
_CH = 16


def _hist_sc_kernel(rec_hbm, c_hbm, rec_v, counts_v, sem0, sem1):
    wid = lax.axis_index("s") * _NC + lax.axis_index("c")
    n_chunks = rec_hbm.shape[0]
    per_w = n_chunks // _NW
    row_iota = lax.broadcasted_iota(jnp.int32, (16,), 0)
    zeros16 = jnp.zeros((16,), jnp.float32)
    ones16 = jnp.ones((16,), jnp.float32)
    sems = [sem0, sem1]

    def do_chunk(ch):
        buf = counts_v.at[ch % 2]
        chunk = wid * per_w + ch
        pltpu.sync_copy(rec_hbm.at[chunk], rec_v)

        def zero_body(i, _):
            off = pl.multiple_of(i * 16, 16)
            for r in range(_CH):
                buf[r, pl.ds(off, 16)] = zeros16
            return 0
        lax.fori_loop(0, _VP // 16, zero_body, 0)

        def scat_body(j, _):
            off = pl.multiple_of(j * 16, 16)
            ids = rec_v[pl.ds(off, 16)]
            plsc.addupdate_scatter(buf, [row_iota, ids], ones16)
            return 0
        lax.fori_loop(0, _KP, scat_body, 0, unroll=4)

        return pltpu.async_copy(
            buf, c_hbm.at[pl.ds(chunk * _CH, _CH)], sems[ch % 2]
        )

    pending = do_chunk(0)
    for ch in range(1, per_w):
        nxt = do_chunk(ch)
        pending.wait()
        pending = nxt
    pending.wait()


def _qf_tc_kernel(c_ref, d_ref, out_ref):
    c = c_ref[...]
    t = jnp.dot(c, d_ref[...], preferred_element_type=jnp.float32)
    out_ref[...] = jnp.sum(t * c, axis=1)


@jax.jit
def kernel(user_sequence, recommendations, distance_matrix):
    del user_sequence
    b, k = recommendations.shape
    v = distance_matrix.shape[0]
    d_pad = jnp.zeros((_VP, _VP), jnp.float32).at[:v, :v].set(distance_matrix)

    rec = recommendations.astype(jnp.int32)
    rec_pad = jnp.full((b, _KP), _DUMMY, jnp.int32).at[:, :k].set(rec)
    rec_t = rec_pad.reshape(b // _CH, _CH, _KP).transpose(0, 2, 1)
    rec_t = rec_t.reshape(b // _CH, _KP * _CH)

    mesh = plsc.VectorSubcoreMesh(core_axis_name="c", subcore_axis_name="s")
    counts = pl.kernel(
        _hist_sc_kernel,
        out_type=jax.ShapeDtypeStruct((b, _VP), jnp.float32),
        mesh=mesh,
        compiler_params=pltpu.CompilerParams(needs_layout_passes=False),
        scratch_types=[
            pltpu.VMEM((_KP * _CH,), jnp.int32),
            pltpu.VMEM((2, _CH, _VP), jnp.float32),
            pltpu.SemaphoreType.DMA,
            pltpu.SemaphoreType.DMA,
        ],
    )(rec_t)

    bblk = 256
    distance_sum = pl.pallas_call(
        _qf_tc_kernel,
        grid=(b // bblk,),
        in_specs=[
            pl.BlockSpec((bblk, _VP), lambda i: (i, 0)),
            pl.BlockSpec((_VP, _VP), lambda i: (0, 0)),
        ],
        out_specs=pl.BlockSpec((bblk,), lambda i: (i,)),
        out_shape=jax.ShapeDtypeStruct((b,), jnp.float32),
    )(counts, d_pad)
    return distance_sum / (k * (k - 1))


# speedup vs baseline: 1.4787x; 1.0541x over previous
import functools

import jax
import jax.numpy as jnp
from jax import lax
from jax.experimental import pallas as pl
from jax.experimental.pallas import tpu as pltpu
from jax.experimental.pallas import tpu_sc as plsc

_VP = 1024
_KP = 64
_DUMMY = 1016
_NC = 2
_NS = 16
_NW = _NC * _NS
_CH = 16


def _hist_sc_kernel(rec_hbm, c_hbm, rec_v, counts_v, sem0, sem1):
    wid = lax.axis_index("s") * _NC + lax.axis_index("c")
    n_chunks = rec_hbm.shape[0]
    per_w = n_chunks // _NW
    row_iota = lax.broadcasted_iota(jnp.int32, (16,), 0)
    zeros16 = jnp.zeros((16,), jnp.float32)
    ones16 = jnp.ones((16,), jnp.float32)
    sems = [sem0, sem1]

    def do_chunk(ch):
        buf = counts_v.at[ch % 2]
        chunk = wid * per_w + ch
        pltpu.sync_copy(rec_hbm.at[chunk], rec_v)

        def zero_body(i, _):
            off = pl.multiple_of(i * 16, 16)
            for r in range(_CH):
                buf[r, pl.ds(off, 16)] = zeros16
            return 0
        lax.fori_loop(0, _VP // 16, zero_body, 0)

        def scat_body(j, _):
            off = pl.multiple_of(j * 16, 16)
            ids = rec_v[pl.ds(off, 16)]
            plsc.addupdate_scatter(buf, [row_iota, ids], ones16)
            return 0
        lax.fori_loop(0, _KP, scat_body, 0, unroll=4)

        return pltpu.async_copy(
            buf, c_hbm.at[pl.ds(chunk * _CH, _CH)], sems[ch % 2]
        )

    pending = do_chunk(0)
    for ch in range(1, per_w):
        nxt = do_chunk(ch)
        pending.wait()
        pending = nxt
    pending.wait()


def _qf_tc_kernel(c_ref, d_ref, out_ref):
    c = c_ref[...]
    t = jnp.dot(c, d_ref[...], preferred_element_type=jnp.float32)
    out_ref[...] = jnp.sum(t * c, axis=1)


@jax.jit
def kernel(user_sequence, recommendations, distance_matrix):
    del user_sequence
    b, k = recommendations.shape
    v = distance_matrix.shape[0]
    d_pad = jnp.zeros((_VP, _VP), jnp.float32).at[:v, :v].set(distance_matrix)

    rec = recommendations.astype(jnp.int32)
    rec_pad = jnp.full((b, _KP), _DUMMY, jnp.int32).at[:, :k].set(rec)
    rec_t = rec_pad.reshape(b // _CH, _CH, _KP).transpose(0, 2, 1)
    rec_t = rec_t.reshape(b // _CH, _KP * _CH)

    mesh = plsc.VectorSubcoreMesh(core_axis_name="c", subcore_axis_name="s")
    n_slices = 2
    bs = b // n_slices
    hist = pl.kernel(
        _hist_sc_kernel,
        out_type=jax.ShapeDtypeStruct((bs, _VP), jnp.float32),
        mesh=mesh,
        compiler_params=pltpu.CompilerParams(needs_layout_passes=False),
        scratch_types=[
            pltpu.VMEM((_KP * _CH,), jnp.int32),
            pltpu.VMEM((2, _CH, _VP), jnp.float32),
            pltpu.SemaphoreType.DMA,
            pltpu.SemaphoreType.DMA,
        ],
    )

    bblk = 256
    qf = functools.partial(
        pl.pallas_call,
        _qf_tc_kernel,
        grid=(bs // bblk,),
        in_specs=[
            pl.BlockSpec((bblk, _VP), lambda i: (i, 0)),
            pl.BlockSpec((_VP, _VP), lambda i: (0, 0)),
        ],
        out_specs=pl.BlockSpec((bblk,), lambda i: (i,)),
        out_shape=jax.ShapeDtypeStruct((bs,), jnp.float32),
    )()

    n_ch_s = bs // _CH
    counts = [hist(rec_t[i * n_ch_s : (i + 1) * n_ch_s]) for i in range(n_slices)]
    sums = [qf(c, d_pad) for c in counts]
    distance_sum = jnp.concatenate(sums)
    return distance_sum / (k * (k - 1))
